# baseline (speedup 1.0000x reference)
import jax
import jax.numpy as jnp
from jax.experimental import pallas as pl

_TEMP_INV = 2.0


def _softmax_block(x_ref, o_ref):
    x = x_ref[...] * _TEMP_INV
    m = jnp.max(x, axis=-1, keepdims=True)
    e = jnp.exp(x - m)
    o_ref[...] = e / jnp.sum(e, axis=-1, keepdims=True)


def kernel(sim, head_idx):
    h, n, k = sim.shape
    rows = h * n
    x = sim.reshape(rows, k)
    block = 2048
    out = pl.pallas_call(
        _softmax_block,
        grid=(rows // block,),
        in_specs=[pl.BlockSpec((block, k), lambda i: (i, 0))],
        out_specs=pl.BlockSpec((block, k), lambda i: (i, 0)),
        out_shape=jax.ShapeDtypeStruct((rows, k), sim.dtype),
    )(x)
    return out.reshape(h, n, k)


# speedup vs baseline: 1.2186x; 1.2186x over previous
import jax
import jax.numpy as jnp
from jax.experimental import pallas as pl

_TEMP_INV = 2.0


def _softmax_block(x_ref, o_ref):
    x = x_ref[...] * _TEMP_INV
    m = jnp.max(x, axis=-1, keepdims=True)
    e = jnp.exp(x - m)
    o_ref[...] = e / jnp.sum(e, axis=-1, keepdims=True)


def kernel(sim, head_idx):
    h, n, k = sim.shape
    block = 2048
    return pl.pallas_call(
        _softmax_block,
        grid=(h, n // block),
        in_specs=[pl.BlockSpec((1, block, k), lambda i, j: (i, j, 0))],
        out_specs=pl.BlockSpec((1, block, k), lambda i, j: (i, j, 0)),
        out_shape=jax.ShapeDtypeStruct((h, n, k), sim.dtype),
    )(sim)


# speedup vs baseline: 1.3944x; 1.1443x over previous
import jax
import jax.numpy as jnp
from jax.experimental import pallas as pl

_TEMP_INV = 2.0


def _softmax_block(x_ref, o_ref):
    x = x_ref[...] * _TEMP_INV
    m = jnp.max(x, axis=-1, keepdims=True)
    e = jnp.exp(x - m)
    o_ref[...] = e / jnp.sum(e, axis=-1, keepdims=True)


def kernel(sim, head_idx):
    h, n, k = sim.shape
    block = 8192
    return pl.pallas_call(
        _softmax_block,
        grid=(h, n // block),
        in_specs=[pl.BlockSpec((1, block, k), lambda i, j: (i, j, 0))],
        out_specs=pl.BlockSpec((1, block, k), lambda i, j: (i, j, 0)),
        out_shape=jax.ShapeDtypeStruct((h, n, k), sim.dtype),
    )(sim)
